# baseline (speedup 1.0000x reference)
import functools
import jax
import jax.numpy as jnp
from jax import lax
from jax.experimental import pallas as pl
from jax.experimental.pallas import tpu as pltpu
from jax.experimental.pallas import tpu_sc as plsc

B = 16384
D = 64
W = D // 2
NW = 32
BPW = B // NW
CB = 128
NCHUNK = BPW // CB
L = 16

_HI = -65536


def _unpack2(x):
  lo = plsc.bitcast(x << 16, jnp.float32)
  hi = plsc.bitcast(x & _HI, jnp.float32)
  return lo, hi


def _tower_kernel(user_table, item_table, uidx_hbm, iidx_hbm, out_hbm,
                  uidx_v, iidx_v, rows_u, rows_v, pbuf, out_v, sem):
  wid = lax.axis_index("s") * 2 + lax.axis_index("c")
  base = wid * BPW

  for c in range(NCHUNK):
    pltpu.sync_copy(uidx_hbm.at[pl.ds(base + c * CB, CB)], uidx_v.at[c])
    pltpu.sync_copy(iidx_hbm.at[pl.ds(base + c * CB, CB)], iidx_v.at[c])
  copies = []
  for c in range(NCHUNK):
    copies.append(pltpu.async_copy(
        user_table.at[uidx_v.at[c]], rows_u.at[pl.ds(c * CB, CB)], sem))
    copies.append(pltpu.async_copy(
        item_table.at[iidx_v.at[c]], rows_v.at[pl.ds(c * CB, CB)], sem))
  for cp in copies:
    cp.wait()

  row_iota = lax.iota(jnp.int32, L)

  def group_body(g, _):
    row0 = pl.multiple_of(g * L, L)
    for k in range(L):
      r = row0 + k
      s = None
      for cc in range(W // L):
        u = rows_u[r, pl.ds(cc * L, L)]
        v = rows_v[r, pl.ds(cc * L, L)]
        ulo, uhi = _unpack2(u)
        vlo, vhi = _unpack2(v)
        m = ulo * vlo + uhi * vhi
        s = m if s is None else s + m
      pbuf[k, :] = s
    acc = jnp.zeros((L,), jnp.float32)
    for l in range(L):
      col = jnp.full((L,), l, jnp.int32)
      acc = acc + plsc.load_gather(pbuf, [row_iota, col])
    out_v[pl.ds(row0, L)] = acc
    return 0

  lax.fori_loop(0, BPW // L, group_body, 0)

  pltpu.sync_copy(out_v, out_hbm.at[pl.ds(base, BPW)])


@jax.jit
def _towers(user_indices, item_indices, user_table, item_table):
  ut = jax.lax.bitcast_convert_type(
      user_table.astype(jnp.bfloat16).reshape(-1, W, 2), jnp.int32)
  it = jax.lax.bitcast_convert_type(
      item_table.astype(jnp.bfloat16).reshape(-1, W, 2), jnp.int32)
  mesh = plsc.VectorSubcoreMesh(core_axis_name="c", subcore_axis_name="s")
  f = pl.kernel(
      _tower_kernel,
      out_type=jax.ShapeDtypeStruct((B,), jnp.float32),
      mesh=mesh,
      compiler_params=pltpu.CompilerParams(
          needs_layout_passes=False, use_tc_tiling_on_sc=False),
      scratch_types=[
          pltpu.VMEM((NCHUNK, CB), jnp.int32),
          pltpu.VMEM((NCHUNK, CB), jnp.int32),
          pltpu.VMEM((BPW, W), jnp.int32),
          pltpu.VMEM((BPW, W), jnp.int32),
          pltpu.VMEM((L, L), jnp.float32),
          pltpu.VMEM((BPW,), jnp.float32),
          pltpu.SemaphoreType.DMA,
      ],
  )
  return f(ut, it, user_indices, item_indices)


def kernel(user_indices, item_indices, user_table, item_table):
  return _towers(user_indices.astype(jnp.int32),
                 item_indices.astype(jnp.int32),
                 user_table, item_table)


# speedup vs baseline: 5.1467x; 5.1467x over previous
import functools
import jax
import jax.numpy as jnp
from jax import lax
from jax.experimental import pallas as pl
from jax.experimental.pallas import tpu as pltpu
from jax.experimental.pallas import tpu_sc as plsc

NU = 1_000_000
B = 16384
D = 64
NW = 32
BPW = B // NW
L = 16
CPG = 256
SH = 8
NGW0 = 124
NGW = 122
NGMAX = 124
TAIL0 = NGW0 * CPG + 31 * NGW * CPG
NVR = B // L
WIN = 32


def _scalar(x):
  return x if jnp.ndim(x) == 0 else x[0]


def _scan_matches(idx_v, lo, hi, lane_iota, per_match):
  def vreg_body(v, _):
    iv = idx_v[pl.ds(v * L, L)]
    mv = jnp.where((iv >= lo) & (iv < hi), jnp.int32(1), jnp.int32(0))
    any0 = _scalar(plsc.all_reduce_population_count(mv != 0))

    def cond(mv):
      return _scalar(plsc.all_reduce_population_count(mv != 0)) > 0

    def body(mv):
      kv = plsc.all_reduce_ffs(mv != 0)
      kv = kv if jnp.ndim(kv) else jnp.full((L,), kv)
      sel = lane_iota == kv
      ival = jnp.sum(jnp.where(sel, iv, 0))
      ks = _scalar(kv)
      per_match(ival, v * L + ks)
      return jnp.where(sel, 0, mv)

    @pl.when(any0 > 0)
    def _():
      lax.while_loop(cond, body, mv)

    return 0

  lax.fori_loop(0, NVR, vreg_body, 0)


def _sweep(tbl, tailr, idx_hbm, out_hbm, idx_v, blk, stage, tailv,
           entry, cnt, off, run, misc, sem, semr, wid, lo, ng, lane_iota):
  hi = lo + ng * CPG
  m0 = lane_iota == 0

  pltpu.sync_copy(idx_hbm, idx_v)

  def zero_body(j, _):
    cnt[j] = 0
    return 0
  lax.fori_loop(0, NGMAX + 2, zero_body, 0)

  def count_match(ival, b):
    g = (ival - lo) >> SH
    cnt[g] = cnt[g] + 1

  _scan_matches(idx_v, lo, hi, lane_iota, count_match)

  off[0] = 0

  def pfx_body(j, _):
    off[j + 1] = off[j] + cnt[j]
    run[j] = off[j]
    return 0
  lax.fori_loop(0, NGMAX, pfx_body, 0)

  def place_match(ival, b):
    il = ival - lo
    g = il >> SH
    p = run[g]
    run[g] = p + 1
    packed = (il << 14) | b
    plsc.store_scatter(entry, [jnp.full((L,), p, jnp.int32)],
                       jnp.full((L,), packed, jnp.int32), mask=m0)

  _scan_matches(idx_v, lo, hi, lane_iota, place_match)

  misc[0] = 0

  def emit_row(bb):
    sc = misc[0]

    @pl.when(sc >= WIN)
    def _():
      pltpu.make_async_copy(stage.at[0], out_hbm.at[0], semr).wait()

    s = sc & (WIN - 1)
    return sc, s, bb

  def finish_row(sc, s, bb):
    pltpu.async_copy(stage.at[s], out_hbm.at[bb], semr)
    misc[0] = sc + 1

  def group_body(j, _):
    @pl.when(j < ng)
    def _():
      pltpu.sync_copy(tbl.at[:, pl.ds(lo + j * CPG, CPG)], blk)
      e0 = off[j]
      e1 = off[j + 1]
      a0 = (e0 >> 4) << 4
      nt = (e1 - a0 + L - 1) >> 4

      def chunk_body(t, _):
        base = a0 + t * L
        pv = entry[pl.ds(base, L)]
        eix = base + lane_iota
        mv = jnp.where((eix >= e0) & (eix < e1), jnp.int32(1), jnp.int32(0))

        def cond(mv):
          return _scalar(plsc.all_reduce_population_count(mv != 0)) > 0

        def body(mv):
          kv = plsc.all_reduce_ffs(mv != 0)
          kv = kv if jnp.ndim(kv) else jnp.full((L,), kv)
          sel = lane_iota == kv
          pk = jnp.sum(jnp.where(sel, pv, 0))
          il = pk >> 14
          bb = pk & 16383
          col = il - j * CPG
          colv = jnp.full((L,), col, jnp.int32)
          sc, s, bb = emit_row(bb)
          for cc in range(D // L):
            u = plsc.load_gather(blk, [lane_iota + cc * L, colv])
            stage[s, pl.ds(cc * L, L)] = u
          finish_row(sc, s, bb)
          return jnp.where(sel, 0, mv)

        lax.while_loop(cond, body, mv)
        return 0

      lax.fori_loop(0, nt, chunk_body, 0)
    return 0

  lax.fori_loop(0, NGMAX, group_body, 0)

  @pl.when(wid == NW - 1)
  def _():
      pltpu.sync_copy(tailr, tailv)

      def tail_match(ival, b):
        ulv = jnp.full((L,), ival - TAIL0, jnp.int32)
        sc, s, bb = emit_row(b)
        for cc in range(D // L):
          u = plsc.load_gather(tailv, [ulv, lane_iota + cc * L])
          stage[s, pl.ds(cc * L, L)] = u
        finish_row(sc, s, bb)

      _scan_matches(idx_v, TAIL0, NU, lane_iota, tail_match)

  ndrain = jnp.minimum(misc[0], WIN)

  def drain_body(t, _):
    pltpu.make_async_copy(stage.at[0], out_hbm.at[0], semr).wait()
    return 0
  lax.fori_loop(0, ndrain, drain_body, 0)


def _gather_kernel(ut, it, utail, itail, uidx, iidx, ug_hbm, vg_hbm,
                   idx_v, blk, stage, tailv, entry, cnt, off, run, misc,
                   sem, semr):
  wid = lax.axis_index("s") * 2 + lax.axis_index("c")
  lane_iota = lax.iota(jnp.int32, L)
  lo = jnp.where(wid == 0, 0, (NGW0 - NGW + NGW * wid) * CPG)
  ng = jnp.where(wid == 0, NGW0, NGW)
  _sweep(ut, utail, uidx, ug_hbm, idx_v, blk, stage, tailv,
         entry, cnt, off, run, misc, sem, semr, wid, lo, ng, lane_iota)
  _sweep(it, itail, iidx, vg_hbm, idx_v, blk, stage, tailv,
         entry, cnt, off, run, misc, sem, semr, wid, lo, ng, lane_iota)


DCH = 128


def _dot_kernel(ug_hbm, vg_hbm, out_hbm, uvm, vvm, pbuf, out_v, sem):
  wid = lax.axis_index("s") * 2 + lax.axis_index("c")
  base = wid * BPW
  row_iota = lax.iota(jnp.int32, L)

  def chunk_body(c, _):
    cb = pl.multiple_of(c * DCH, DCH)
    cu = pltpu.async_copy(ug_hbm.at[pl.ds(base + cb, DCH)], uvm, sem)
    cv = pltpu.async_copy(vg_hbm.at[pl.ds(base + cb, DCH)], vvm, sem)
    cu.wait()
    cv.wait()

    def group_body(g, _):
      row0 = pl.multiple_of(g * L, L)
      for k in range(L):
        r = row0 + k
        s = None
        for cc in range(D // L):
          u = uvm[r, pl.ds(cc * L, L)]
          v = vvm[r, pl.ds(cc * L, L)]
          m = u * v
          s = m if s is None else s + m
        pbuf[k, :] = s
      acc = jnp.zeros((L,), jnp.float32)
      for l in range(L):
        col = jnp.full((L,), l, jnp.int32)
        acc = acc + plsc.load_gather(pbuf, [row_iota, col])
      out_v[pl.ds(cb + row0, L)] = acc
      return 0

    lax.fori_loop(0, DCH // L, group_body, 0)
    return 0

  lax.fori_loop(0, BPW // DCH, chunk_body, 0)
  pltpu.sync_copy(out_v, out_hbm.at[pl.ds(base, BPW)])


@jax.jit
def _towers(user_indices, item_indices, user_table, item_table):
  mesh = plsc.VectorSubcoreMesh(core_axis_name="c", subcore_axis_name="s")
  k1 = pl.kernel(
      _gather_kernel,
      out_type=(pltpu.HBM((B, D), jnp.float32),
                pltpu.HBM((B, D), jnp.float32)),
      mesh=mesh,
      compiler_params=pltpu.CompilerParams(needs_layout_passes=False),
      scratch_types=[
          pltpu.VMEM((B,), jnp.int32),
          pltpu.VMEM((D, CPG), jnp.float32),
          pltpu.VMEM((WIN, D), jnp.float32),
          pltpu.VMEM((D, D), jnp.float32),
          pltpu.VMEM((B + L,), jnp.int32),
          pltpu.SMEM((NGMAX + 2,), jnp.int32),
          pltpu.SMEM((NGMAX + 2,), jnp.int32),
          pltpu.SMEM((NGMAX + 2,), jnp.int32),
          pltpu.SMEM((8,), jnp.int32),
          pltpu.SemaphoreType.DMA,
          pltpu.SemaphoreType.DMA,
      ],
  )
  ug, vg = k1(user_table.T, item_table.T,
              user_table[TAIL0:], item_table[TAIL0:],
              user_indices, item_indices)
  k2 = pl.kernel(
      _dot_kernel,
      out_type=jax.ShapeDtypeStruct((B,), jnp.float32),
      mesh=mesh,
      compiler_params=pltpu.CompilerParams(needs_layout_passes=False),
      scratch_types=[
          pltpu.VMEM((DCH, D), jnp.float32),
          pltpu.VMEM((DCH, D), jnp.float32),
          pltpu.VMEM((L, L), jnp.float32),
          pltpu.VMEM((BPW,), jnp.float32),
          pltpu.SemaphoreType.DMA,
      ],
  )
  return k2(ug, vg)


def kernel(user_indices, item_indices, user_table, item_table):
  return _towers(user_indices.astype(jnp.int32),
                 item_indices.astype(jnp.int32),
                 user_table, item_table)


# speedup vs baseline: 6.1628x; 1.1974x over previous
import functools
import jax
import jax.numpy as jnp
from jax import lax
from jax.experimental import pallas as pl
from jax.experimental.pallas import tpu as pltpu
from jax.experimental.pallas import tpu_sc as plsc

NU = 1_000_000
B = 16384
D = 64
NW = 32
BPW = B // NW
L = 16
CPG = 128
SH = 7
NGW = 244
NGX = 4
NGMAX = 245
TAIL0 = (NW * NGW + NGX) * CPG
NVR = B // L
WIN = 32


def _scalar(x):
  return x if jnp.ndim(x) == 0 else x[0]


def _scan_matches(idx_v, lo, hi, lane_iota, per_match):
  def vreg_body(v, _):
    iv = idx_v[pl.ds(v * L, L)]
    mv = jnp.where((iv >= lo) & (iv < hi), jnp.int32(1), jnp.int32(0))
    any0 = _scalar(plsc.all_reduce_population_count(mv != 0))

    def cond(mv):
      return _scalar(plsc.all_reduce_population_count(mv != 0)) > 0

    def body(mv):
      kv = plsc.all_reduce_ffs(mv != 0)
      kv = kv if jnp.ndim(kv) else jnp.full((L,), kv)
      sel = lane_iota == kv
      ival = jnp.sum(jnp.where(sel, iv, 0))
      ks = _scalar(kv)
      per_match(ival, v * L + ks)
      return jnp.where(sel, 0, mv)

    @pl.when(any0 > 0)
    def _():
      lax.while_loop(cond, body, mv)

    return 0

  lax.fori_loop(0, NVR, vreg_body, 0)


def _sweep(tbl, tailr, idx_hbm, out_hbm, idx_v, blk, stage, tailv,
           entry, cnt, off, run, misc, sem, semr, wid, lo, ng, lane_iota):
  hi = lo + ng * CPG
  m0 = lane_iota == 0

  pltpu.sync_copy(idx_hbm, idx_v)

  def zero_body(j, _):
    cnt[j] = 0
    return 0
  lax.fori_loop(0, NGMAX + 2, zero_body, 0)

  def count_match(ival, b):
    g = (ival - lo) >> SH
    cnt[g] = cnt[g] + 1

  _scan_matches(idx_v, lo, hi, lane_iota, count_match)

  off[0] = 0

  def pfx_body(j, _):
    off[j + 1] = off[j] + cnt[j]
    run[j] = off[j]
    return 0
  lax.fori_loop(0, NGMAX, pfx_body, 0)

  def place_match(ival, b):
    il = ival - lo
    g = il >> SH
    p = run[g]
    run[g] = p + 1
    packed = (il << 14) | b
    plsc.store_scatter(entry, [jnp.full((L,), p, jnp.int32)],
                       jnp.full((L,), packed, jnp.int32), mask=m0)

  _scan_matches(idx_v, lo, hi, lane_iota, place_match)

  misc[0] = 0

  def emit_row(bb):
    sc = misc[0]

    @pl.when(sc >= WIN)
    def _():
      pltpu.make_async_copy(stage.at[0], out_hbm.at[0], semr).wait()

    s = sc & (WIN - 1)
    return sc, s, bb

  def finish_row(sc, s, bb):
    pltpu.async_copy(stage.at[s], out_hbm.at[bb], semr)
    misc[0] = sc + 1

  def issue_group(j):
    pltpu.async_copy(tbl.at[:, pl.ds(lo + j * CPG, CPG)],
                     blk.at[j & 1], sem)

  @pl.when(cnt[0] > 0)
  def _():
    issue_group(0)

  def group_body(j, _):
    @pl.when((j + 1 < ng) & (cnt[j + 1] > 0))
    def _():
      issue_group(j + 1)

    @pl.when((j < ng) & (cnt[j] > 0))
    def _():
      pltpu.make_async_copy(tbl.at[:, pl.ds(0, CPG)], blk.at[0],
                            sem).wait()
      bufv = jnp.full((L,), j & 1, jnp.int32)
      e0 = off[j]
      e1 = off[j + 1]
      a0 = (e0 >> 4) << 4
      nt = (e1 - a0 + L - 1) >> 4

      def chunk_body(t, _):
        base = a0 + t * L
        pv = entry[pl.ds(base, L)]
        eix = base + lane_iota
        mv = jnp.where((eix >= e0) & (eix < e1), jnp.int32(1), jnp.int32(0))

        def cond(mv):
          return _scalar(plsc.all_reduce_population_count(mv != 0)) > 0

        def body(mv):
          kv = plsc.all_reduce_ffs(mv != 0)
          kv = kv if jnp.ndim(kv) else jnp.full((L,), kv)
          sel = lane_iota == kv
          pk = jnp.sum(jnp.where(sel, pv, 0))
          il = pk >> 14
          bb = pk & 16383
          col = il - j * CPG
          colv = jnp.full((L,), col, jnp.int32)
          sc, s, bb = emit_row(bb)
          for cc in range(D // L):
            u = plsc.load_gather(blk, [bufv, lane_iota + cc * L, colv])
            stage[s, pl.ds(cc * L, L)] = u
          finish_row(sc, s, bb)
          return jnp.where(sel, 0, mv)

        lax.while_loop(cond, body, mv)
        return 0

      lax.fori_loop(0, nt, chunk_body, 0)
    return 0

  lax.fori_loop(0, NGMAX, group_body, 0)

  @pl.when(wid == NW - 1)
  def _():
      pltpu.sync_copy(tailr, tailv)

      def tail_match(ival, b):
        ulv = jnp.full((L,), ival - TAIL0, jnp.int32)
        sc, s, bb = emit_row(b)
        for cc in range(D // L):
          u = plsc.load_gather(tailv, [ulv, lane_iota + cc * L])
          stage[s, pl.ds(cc * L, L)] = u
        finish_row(sc, s, bb)

      _scan_matches(idx_v, TAIL0, NU, lane_iota, tail_match)

  ndrain = jnp.minimum(misc[0], WIN)

  def drain_body(t, _):
    pltpu.make_async_copy(stage.at[0], out_hbm.at[0], semr).wait()
    return 0
  lax.fori_loop(0, ndrain, drain_body, 0)


def _gather_kernel(ut, it, utail, itail, uidx, iidx, ug_hbm, vg_hbm,
                   idx_v, blk, stage, tailv, entry, cnt, off, run, misc,
                   sem, semr):
  wid = lax.axis_index("s") * 2 + lax.axis_index("c")
  lane_iota = lax.iota(jnp.int32, L)
  lo = (NGW * wid + jnp.minimum(wid, NGX)) * CPG
  ng = jnp.where(wid < NGX, NGW + 1, NGW)
  _sweep(ut, utail, uidx, ug_hbm, idx_v, blk, stage, tailv,
         entry, cnt, off, run, misc, sem, semr, wid, lo, ng, lane_iota)
  _sweep(it, itail, iidx, vg_hbm, idx_v, blk, stage, tailv,
         entry, cnt, off, run, misc, sem, semr, wid, lo, ng, lane_iota)


DCH = 128


def _dot_kernel(ug_hbm, vg_hbm, out_hbm, uvm, vvm, pbuf, out_v, sem):
  wid = lax.axis_index("s") * 2 + lax.axis_index("c")
  base = wid * BPW
  row_iota = lax.iota(jnp.int32, L)

  def chunk_body(c, _):
    cb = pl.multiple_of(c * DCH, DCH)
    cu = pltpu.async_copy(ug_hbm.at[pl.ds(base + cb, DCH)], uvm, sem)
    cv = pltpu.async_copy(vg_hbm.at[pl.ds(base + cb, DCH)], vvm, sem)
    cu.wait()
    cv.wait()

    def group_body(g, _):
      row0 = pl.multiple_of(g * L, L)
      for k in range(L):
        r = row0 + k
        s = None
        for cc in range(D // L):
          u = uvm[r, pl.ds(cc * L, L)]
          v = vvm[r, pl.ds(cc * L, L)]
          m = u * v
          s = m if s is None else s + m
        pbuf[k, :] = s
      acc = jnp.zeros((L,), jnp.float32)
      for l in range(L):
        col = jnp.full((L,), l, jnp.int32)
        acc = acc + plsc.load_gather(pbuf, [row_iota, col])
      out_v[pl.ds(cb + row0, L)] = acc
      return 0

    lax.fori_loop(0, DCH // L, group_body, 0)
    return 0

  lax.fori_loop(0, BPW // DCH, chunk_body, 0)
  pltpu.sync_copy(out_v, out_hbm.at[pl.ds(base, BPW)])


@jax.jit
def _towers(user_indices, item_indices, user_table, item_table):
  mesh = plsc.VectorSubcoreMesh(core_axis_name="c", subcore_axis_name="s")
  k1 = pl.kernel(
      _gather_kernel,
      out_type=(pltpu.HBM((B, D), jnp.float32),
                pltpu.HBM((B, D), jnp.float32)),
      mesh=mesh,
      compiler_params=pltpu.CompilerParams(needs_layout_passes=False),
      scratch_types=[
          pltpu.VMEM((B,), jnp.int32),
          pltpu.VMEM((2, D, CPG), jnp.float32),
          pltpu.VMEM((WIN, D), jnp.float32),
          pltpu.VMEM((D, D), jnp.float32),
          pltpu.VMEM((B + L,), jnp.int32),
          pltpu.SMEM((NGMAX + 2,), jnp.int32),
          pltpu.SMEM((NGMAX + 2,), jnp.int32),
          pltpu.SMEM((NGMAX + 2,), jnp.int32),
          pltpu.SMEM((8,), jnp.int32),
          pltpu.SemaphoreType.DMA,
          pltpu.SemaphoreType.DMA,
      ],
  )
  ug, vg = k1(user_table.T, item_table.T,
              user_table[TAIL0:], item_table[TAIL0:],
              user_indices, item_indices)
  k2 = pl.kernel(
      _dot_kernel,
      out_type=jax.ShapeDtypeStruct((B,), jnp.float32),
      mesh=mesh,
      compiler_params=pltpu.CompilerParams(needs_layout_passes=False),
      scratch_types=[
          pltpu.VMEM((DCH, D), jnp.float32),
          pltpu.VMEM((DCH, D), jnp.float32),
          pltpu.VMEM((L, L), jnp.float32),
          pltpu.VMEM((BPW,), jnp.float32),
          pltpu.SemaphoreType.DMA,
      ],
  )
  return k2(ug, vg)


def kernel(user_indices, item_indices, user_table, item_table):
  return _towers(user_indices.astype(jnp.int32),
                 item_indices.astype(jnp.int32),
                 user_table, item_table)


# speedup vs baseline: 6.2757x; 1.0183x over previous
import functools
import jax
import jax.numpy as jnp
from jax import lax
from jax.experimental import pallas as pl
from jax.experimental.pallas import tpu as pltpu
from jax.experimental.pallas import tpu_sc as plsc

NU = 1_000_000
B = 16384
D = 64
NW = 32
BPW = B // NW
L = 16
CPG = 128
SH = 7
NGW = 244
NGX = 4
NGMAX = 245
TAIL0 = (NW * NGW + NGX) * CPG
NVR = B // L
NSB = 4
GW = 128


def _scalar(x):
  return x if jnp.ndim(x) == 0 else x[0]


def _scan_matches(idx_v, lo, hi, lane_iota, per_match):
  def vreg_body(v, _):
    iv = idx_v[pl.ds(v * L, L)]
    mv = jnp.where((iv >= lo) & (iv < hi), jnp.int32(1), jnp.int32(0))
    any0 = _scalar(plsc.all_reduce_population_count(mv != 0))

    def cond(mv):
      return _scalar(plsc.all_reduce_population_count(mv != 0)) > 0

    def body(mv):
      kv = plsc.all_reduce_ffs(mv != 0)
      kv = kv if jnp.ndim(kv) else jnp.full((L,), kv)
      sel = lane_iota == kv
      ival = jnp.sum(jnp.where(sel, iv, 0))
      ks = _scalar(kv)
      per_match(ival, v * L + ks)
      return jnp.where(sel, 0, mv)

    @pl.when(any0 > 0)
    def _():
      lax.while_loop(cond, body, mv)

    return 0

  lax.fori_loop(0, NVR, vreg_body, 0)


def _sweep(tbl, tailr, idx_hbm, out_hbm, idx_v, blk, stage, bidx, tailv,
           entry, cnt, off, run, misc, sem, semr, wid, lo, ng, lane_iota):
  hi = lo + ng * CPG
  m0 = lane_iota == 0

  pltpu.sync_copy(idx_hbm, idx_v)

  def zero_body(j, _):
    cnt[j] = 0
    return 0
  lax.fori_loop(0, NGMAX + 2, zero_body, 0)

  def count_match(ival, b):
    g = (ival - lo) >> SH
    cnt[g] = cnt[g] + 1

  _scan_matches(idx_v, lo, hi, lane_iota, count_match)

  off[0] = 0

  def pfx_body(j, _):
    off[j + 1] = off[j] + cnt[j]
    run[j] = off[j]
    return 0
  lax.fori_loop(0, NGMAX, pfx_body, 0)

  def place_match(ival, b):
    il = ival - lo
    g = il >> SH
    p = run[g]
    run[g] = p + 1
    packed = (il << 14) | b
    plsc.store_scatter(entry, [jnp.full((L,), p, jnp.int32)],
                       jnp.full((L,), packed, jnp.int32), mask=m0)

  _scan_matches(idx_v, lo, hi, lane_iota, place_match)

  misc[0] = 0

  def emit_begin():
    fc = misc[0]
    r = (fc >> 4) & (NSB - 1)
    p = fc & (L - 1)

    @pl.when((p == 0) & (fc >= NSB * L))
    def _():
      pltpu.make_async_copy(stage.at[0], out_hbm.at[bidx.at[0]],
                            semr).wait()

    return fc, r, p

  def emit_end(fc, r, p, bb):
    plsc.store_scatter(bidx.at[r], [jnp.full((L,), p, jnp.int32)],
                       jnp.full((L,), bb, jnp.int32), mask=m0)
    misc[0] = fc + 1

    @pl.when(p == L - 1)
    def _():
      pltpu.async_copy(stage.at[r], out_hbm.at[bidx.at[r]], semr)

  def issue_group(j):
    pltpu.async_copy(tbl.at[:, pl.ds(lo + j * CPG, CPG)],
                     blk.at[j & 1], sem)

  @pl.when(cnt[0] > 0)
  def _():
    issue_group(0)

  def group_body(j, _):
    @pl.when((j + 1 < ng) & (cnt[j + 1] > 0))
    def _():
      issue_group(j + 1)

    @pl.when((j < ng) & (cnt[j] > 0))
    def _():
      pltpu.make_async_copy(tbl.at[:, pl.ds(0, CPG)], blk.at[0],
                            sem).wait()
      bufv = jnp.full((L,), j & 1, jnp.int32)
      e0 = off[j]
      e1 = off[j + 1]

      def entry_body(e, _):
        pa = (e >> 4) << 4
        pv = entry[pl.ds(pa, L)]
        pk = jnp.take(pv, jnp.full((L,), e - pa, jnp.int32))[0]
        il = pk >> 14
        bb = pk & 16383
        colv = jnp.full((L,), il - j * CPG, jnp.int32)
        fc, r, p = emit_begin()
        for cc in range(D // L):
          u = plsc.load_gather(blk, [bufv, lane_iota + cc * L, colv])
          stage[r, p, pl.ds(cc * L, L)] = u
        emit_end(fc, r, p, bb)
        return 0

      lax.fori_loop(e0, e1, entry_body, 0)
    return 0

  lax.fori_loop(0, NGMAX, group_body, 0)

  @pl.when(wid == NW - 1)
  def _():
      pltpu.sync_copy(tailr, tailv)

      def tail_match(ival, b):
        ulv = jnp.full((L,), ival - TAIL0, jnp.int32)
        fc, r, p = emit_begin()
        for cc in range(D // L):
          u = plsc.load_gather(tailv, [ulv, lane_iota + cc * L])
          stage[r, p, pl.ds(cc * L, L)] = u
        emit_end(fc, r, p, b)

      _scan_matches(idx_v, TAIL0, NU, lane_iota, tail_match)

  fcf = misc[0]
  ppf = fcf & (L - 1)
  rrf = (fcf >> 4) & (NSB - 1)

  @pl.when(ppf > 0)
  def _():
    brow = bidx[rrf, pl.ds(0, L)]
    brow = jnp.where(lane_iota >= ppf, B + lane_iota, brow)
    bidx[rrf, pl.ds(0, L)] = brow
    pltpu.async_copy(stage.at[rrf], out_hbm.at[bidx.at[rrf]], semr)

  ndrain = jnp.minimum((fcf + L - 1) >> 4, NSB)

  def drain_body(t, _):
    pltpu.make_async_copy(stage.at[0], out_hbm.at[bidx.at[0]], semr).wait()
    return 0
  lax.fori_loop(0, ndrain, drain_body, 0)


def _gather_kernel(ut, it, utail, itail, uidx, iidx, ug_hbm, vg_hbm,
                   idx_v, blk, stage, bidx, tailv, entry, cnt, off, run,
                   misc, sem, semr):
  wid = lax.axis_index("s") * 2 + lax.axis_index("c")
  lane_iota = lax.iota(jnp.int32, L)
  lo = (NGW * wid + jnp.minimum(wid, NGX)) * CPG
  ng = jnp.where(wid < NGX, NGW + 1, NGW)
  _sweep(ut, utail, uidx, ug_hbm, idx_v, blk, stage, bidx, tailv,
         entry, cnt, off, run, misc, sem, semr, wid, lo, ng, lane_iota)
  _sweep(it, itail, iidx, vg_hbm, idx_v, blk, stage, bidx, tailv,
         entry, cnt, off, run, misc, sem, semr, wid, lo, ng, lane_iota)


DCH = 128


def _dot_kernel(ug_hbm, vg_hbm, out_hbm, uvm, vvm, pbuf, out_v, sem):
  wid = lax.axis_index("s") * 2 + lax.axis_index("c")
  base = wid * BPW
  row_iota = lax.iota(jnp.int32, L)

  def chunk_body(c, _):
    cb = pl.multiple_of(c * DCH, DCH)
    cu = pltpu.async_copy(ug_hbm.at[pl.ds(base + cb, DCH)], uvm, sem)
    cv = pltpu.async_copy(vg_hbm.at[pl.ds(base + cb, DCH)], vvm, sem)
    cu.wait()
    cv.wait()

    def group_body(g, _):
      row0 = pl.multiple_of(g * L, L)
      for k in range(L):
        r = row0 + k
        s = None
        for cc in range(D // L):
          u = uvm[r, pl.ds(cc * L, L)]
          v = vvm[r, pl.ds(cc * L, L)]
          m = u * v
          s = m if s is None else s + m
        pbuf[k, :] = s
      acc = jnp.zeros((L,), jnp.float32)
      for l in range(L):
        col = jnp.full((L,), l, jnp.int32)
        acc = acc + plsc.load_gather(pbuf, [row_iota, col])
      out_v[pl.ds(cb + row0, L)] = acc
      return 0

    lax.fori_loop(0, DCH // L, group_body, 0)
    return 0

  lax.fori_loop(0, BPW // DCH, chunk_body, 0)
  pltpu.sync_copy(out_v, out_hbm.at[pl.ds(base, BPW)])


@jax.jit
def _towers(user_indices, item_indices, user_table, item_table):
  mesh = plsc.VectorSubcoreMesh(core_axis_name="c", subcore_axis_name="s")
  k1 = pl.kernel(
      _gather_kernel,
      out_type=(pltpu.HBM((B + L, GW), jnp.float32),
                pltpu.HBM((B + L, GW), jnp.float32)),
      mesh=mesh,
      compiler_params=pltpu.CompilerParams(needs_layout_passes=False),
      scratch_types=[
          pltpu.VMEM((B,), jnp.int32),
          pltpu.VMEM((2, D, CPG), jnp.float32),
          pltpu.VMEM((NSB, L, GW), jnp.float32),
          pltpu.VMEM((NSB, L), jnp.int32),
          pltpu.VMEM((D, D), jnp.float32),
          pltpu.VMEM((B + L,), jnp.int32),
          pltpu.SMEM((NGMAX + 2,), jnp.int32),
          pltpu.SMEM((NGMAX + 2,), jnp.int32),
          pltpu.SMEM((NGMAX + 2,), jnp.int32),
          pltpu.SMEM((8,), jnp.int32),
          pltpu.SemaphoreType.DMA,
          pltpu.SemaphoreType.DMA,
      ],
  )
  ug, vg = k1(user_table.T, item_table.T,
              user_table[TAIL0:], item_table[TAIL0:],
              user_indices, item_indices)
  k2 = pl.kernel(
      _dot_kernel,
      out_type=jax.ShapeDtypeStruct((B,), jnp.float32),
      mesh=mesh,
      compiler_params=pltpu.CompilerParams(needs_layout_passes=False),
      scratch_types=[
          pltpu.VMEM((DCH, GW), jnp.float32),
          pltpu.VMEM((DCH, GW), jnp.float32),
          pltpu.VMEM((L, L), jnp.float32),
          pltpu.VMEM((BPW,), jnp.float32),
          pltpu.SemaphoreType.DMA,
      ],
  )
  return k2(ug, vg)


def kernel(user_indices, item_indices, user_table, item_table):
  return _towers(user_indices.astype(jnp.int32),
                 item_indices.astype(jnp.int32),
                 user_table, item_table)


# speedup vs baseline: 7.6447x; 1.2182x over previous
import functools
import jax
import jax.numpy as jnp
from jax import lax
from jax.experimental import pallas as pl
from jax.experimental.pallas import tpu as pltpu
from jax.experimental.pallas import tpu_sc as plsc

NU = 1_000_000
B = 16384
D = 64
NW = 32
BPW = B // NW
L = 16
CPG = 128
SH = 7
NGW = 244
NGX = 4
NGMAX = 245
NBIN = 256
TAIL0 = (NW * NGW + NGX) * CPG
NVR = B // L
NSB = 4
GW = 128
DCH = 128


def _scalar(x):
  return x if jnp.ndim(x) == 0 else x[0]


def _splat(x):
  return x if jnp.ndim(x) else jnp.full((L,), x)


def _sweep(tbl, tailr, idx_hbm, out_hbm, idx_v, blk, stage, bidx, tailv,
           entry, scrv, cntv, runv, confv, cnt, off, misc, sem, semr,
           wid, lo, ng, lane_iota):
  hi_scan = jnp.where(wid == NW - 1, NU, lo + ng * CPG)
  m0 = lane_iota == 0
  ones = jnp.ones((L,), jnp.int32)

  pltpu.sync_copy(idx_hbm, idx_v)

  for k in range(NBIN // L):
    cntv[pl.ds(k * L, L)] = jnp.zeros((L,), jnp.int32)

  def fast_pass(apply_vec):
    def vbody(v, _):
      iv = idx_v[pl.ds(v * L, L)]
      m = (iv >= lo) & (iv < hi_scan)
      il = iv - lo
      g = jnp.where(m, il >> SH, NGMAX + 1)
      plsc.store_scatter(scrv, [g], lane_iota)
      rb = plsc.load_gather(scrv, [g])
      dup = m & (rb != lane_iota)
      dupc = _splat(plsc.all_reduce_population_count(dup))
      ok = m & (dupc == 0)
      apply_vec(g, il, v * L + lane_iota, ok)
      plsc.store_scatter(confv, [jnp.full((L,), v, jnp.int32)], dupc,
                         mask=m0)
      return 0

    lax.fori_loop(0, NVR, vbody, 0)

  def repair_pass(apply_vec):
    def qbody(q, _):
      cv = confv[pl.ds(q * L, L)]
      s = _scalar(plsc.all_reduce_population_count(cv != 0))

      @pl.when(s > 0)
      def _():
        def cond(mq):
          return _scalar(plsc.all_reduce_population_count(mq != 0)) > 0

        def wbody(mq):
          kq = _splat(plsc.all_reduce_ffs(mq != 0))
          v = q * L + _scalar(kq)
          iv = idx_v[pl.ds(v * L, L)]
          m2 = (iv >= lo) & (iv < hi_scan)
          il2 = iv - lo
          g2 = jnp.where(m2, il2 >> SH, NGMAX + 1)
          bv2 = v * L + lane_iota

          def cond2(mm):
            return _scalar(plsc.all_reduce_population_count(mm != 0)) > 0

          def wbody2(mm):
            k2 = _splat(plsc.all_reduce_ffs(mm != 0))
            sel = lane_iota == k2
            apply_vec(g2, il2, bv2, sel & m2)
            return jnp.where(sel, 0, mm)

          lax.while_loop(cond2, wbody2,
                         jnp.where(m2, jnp.int32(1), jnp.int32(0)))
          return jnp.where(lane_iota == kq, 0, mq)

        lax.while_loop(cond, wbody, cv)

      return 0

    lax.fori_loop(0, NVR // L, qbody, 0)

  def count_vec(g, il, bvec, ok):
    plsc.addupdate_scatter(cntv, [g], ones, mask=ok)

  fast_pass(count_vec)
  repair_pass(count_vec)

  tot = jnp.zeros((L,), jnp.int32)
  lastv = jnp.full((L,), L - 1, jnp.int32)
  for vb in range(NBIN // L):
    c = cntv[pl.ds(vb * L, L)]
    pc = plsc.cumsum(c)
    ex = tot + pc - c
    runv[pl.ds(vb * L, L)] = ex
    for k in range(L):
      off[vb * L + k] = ex[k]
      cnt[vb * L + k] = c[k]
    tot = tot + jnp.take(pc, lastv)

  def place_vec(g, il, bvec, ok):
    pos = plsc.load_gather(runv, [g])
    packed = (il << 14) | bvec
    plsc.store_scatter(entry, [pos], packed, mask=ok)
    plsc.addupdate_scatter(runv, [g], ones, mask=ok)

  fast_pass(place_vec)
  repair_pass(place_vec)

  misc[0] = 0

  def emit_begin():
    fc = misc[0]
    r = (fc >> 4) & (NSB - 1)
    p = fc & (L - 1)

    @pl.when((p == 0) & (fc >= NSB * L))
    def _():
      pltpu.make_async_copy(stage.at[0], out_hbm.at[bidx.at[0]],
                            semr).wait()

    return fc, r, p

  def emit_end(fc, r, p, bb):
    plsc.store_scatter(bidx.at[r], [jnp.full((L,), p, jnp.int32)],
                       jnp.full((L,), bb, jnp.int32), mask=m0)
    misc[0] = fc + 1

    @pl.when(p == L - 1)
    def _():
      pltpu.async_copy(stage.at[r], out_hbm.at[bidx.at[r]], semr)

  def issue_group(j):
    pltpu.async_copy(tbl.at[:, pl.ds(lo + j * CPG, CPG)],
                     blk.at[j & 1], sem)

  @pl.when(cnt[0] > 0)
  def _():
    issue_group(0)

  def group_body(j, _):
    @pl.when((j + 1 < ng) & (cnt[j + 1] > 0))
    def _():
      issue_group(j + 1)

    @pl.when((j < ng) & (cnt[j] > 0))
    def _():
      pltpu.make_async_copy(tbl.at[:, pl.ds(0, CPG)], blk.at[0],
                            sem).wait()
      bufv = jnp.full((L,), j & 1, jnp.int32)
      e0 = off[j]
      e1 = off[j + 1]

      def entry_body(e, _):
        pa = (e >> 4) << 4
        pv = entry[pl.ds(pa, L)]
        pk = jnp.take(pv, jnp.full((L,), e - pa, jnp.int32))[0]
        il = pk >> 14
        bb = pk & 16383
        colv = jnp.full((L,), il - j * CPG, jnp.int32)
        fc, r, p = emit_begin()
        for cc in range(D // L):
          u = plsc.load_gather(blk, [bufv, lane_iota + cc * L, colv])
          stage[r, p, pl.ds(cc * L, L)] = u
        emit_end(fc, r, p, bb)
        return 0

      lax.fori_loop(e0, e1, entry_body, 0)
    return 0

  lax.fori_loop(0, NGMAX, group_body, 0)

  @pl.when(wid == NW - 1)
  def _():
    pltpu.sync_copy(tailr, tailv)
    e0 = off[NGW]
    e1 = off[NGW + 1]

    def tail_entry(e, _):
      pa = (e >> 4) << 4
      pv = entry[pl.ds(pa, L)]
      pk = jnp.take(pv, jnp.full((L,), e - pa, jnp.int32))[0]
      il = pk >> 14
      bb = pk & 16383
      ulv = jnp.full((L,), il - NGW * CPG, jnp.int32)
      fc, r, p = emit_begin()
      for cc in range(D // L):
        u = plsc.load_gather(tailv, [ulv, lane_iota + cc * L])
        stage[r, p, pl.ds(cc * L, L)] = u
      emit_end(fc, r, p, bb)
      return 0

    lax.fori_loop(e0, e1, tail_entry, 0)

  fcf = misc[0]
  ppf = fcf & (L - 1)
  rrf = (fcf >> 4) & (NSB - 1)

  @pl.when(ppf > 0)
  def _():
    brow = bidx[rrf, pl.ds(0, L)]
    brow = jnp.where(lane_iota >= ppf, B + lane_iota, brow)
    bidx[rrf, pl.ds(0, L)] = brow
    pltpu.async_copy(stage.at[rrf], out_hbm.at[bidx.at[rrf]], semr)

  ndrain = jnp.minimum((fcf + L - 1) >> 4, NSB)

  def drain_body(t, _):
    pltpu.make_async_copy(stage.at[0], out_hbm.at[bidx.at[0]], semr).wait()
    return 0
  lax.fori_loop(0, ndrain, drain_body, 0)


def _gather_kernel(ut, it, utail, itail, uidx, iidx, ug_hbm, vg_hbm,
                   idx_v, blk, stage, bidx, tailv, entry, scrv, cntv,
                   runv, confv, cnt, off, misc, sem, semr):
  wid = lax.axis_index("s") * 2 + lax.axis_index("c")
  lane_iota = lax.iota(jnp.int32, L)
  lo = (NGW * wid + jnp.minimum(wid, NGX)) * CPG
  ng = jnp.where(wid < NGX, NGW + 1, NGW)
  _sweep(ut, utail, uidx, ug_hbm, idx_v, blk, stage, bidx, tailv,
         entry, scrv, cntv, runv, confv, cnt, off, misc, sem, semr,
         wid, lo, ng, lane_iota)
  _sweep(it, itail, iidx, vg_hbm, idx_v, blk, stage, bidx, tailv,
         entry, scrv, cntv, runv, confv, cnt, off, misc, sem, semr,
         wid, lo, ng, lane_iota)


def _dot_kernel(ug_hbm, vg_hbm, out_hbm, uvm, vvm, pbuf, out_v, sem):
  wid = lax.axis_index("s") * 2 + lax.axis_index("c")
  base = wid * BPW
  row_iota = lax.iota(jnp.int32, L)

  def chunk_body(c, _):
    cb = pl.multiple_of(c * DCH, DCH)
    cu = pltpu.async_copy(ug_hbm.at[pl.ds(base + cb, DCH)], uvm, sem)
    cv = pltpu.async_copy(vg_hbm.at[pl.ds(base + cb, DCH)], vvm, sem)
    cu.wait()
    cv.wait()

    def group_body(g, _):
      row0 = pl.multiple_of(g * L, L)
      for k in range(L):
        r = row0 + k
        s = None
        for cc in range(D // L):
          u = uvm[r, pl.ds(cc * L, L)]
          v = vvm[r, pl.ds(cc * L, L)]
          m = u * v
          s = m if s is None else s + m
        pbuf[k, :] = s
      acc = jnp.zeros((L,), jnp.float32)
      for l in range(L):
        col = jnp.full((L,), l, jnp.int32)
        acc = acc + plsc.load_gather(pbuf, [row_iota, col])
      out_v[pl.ds(cb + row0, L)] = acc
      return 0

    lax.fori_loop(0, DCH // L, group_body, 0)
    return 0

  lax.fori_loop(0, BPW // DCH, chunk_body, 0)
  pltpu.sync_copy(out_v, out_hbm.at[pl.ds(base, BPW)])


@jax.jit
def _towers(user_indices, item_indices, user_table, item_table):
  mesh = plsc.VectorSubcoreMesh(core_axis_name="c", subcore_axis_name="s")
  k1 = pl.kernel(
      _gather_kernel,
      out_type=(pltpu.HBM((B + L, GW), jnp.float32),
                pltpu.HBM((B + L, GW), jnp.float32)),
      mesh=mesh,
      compiler_params=pltpu.CompilerParams(needs_layout_passes=False),
      scratch_types=[
          pltpu.VMEM((B,), jnp.int32),
          pltpu.VMEM((2, D, CPG), jnp.float32),
          pltpu.VMEM((NSB, L, GW), jnp.float32),
          pltpu.VMEM((NSB, L), jnp.int32),
          pltpu.VMEM((D, D), jnp.float32),
          pltpu.VMEM((B + L,), jnp.int32),
          pltpu.VMEM((NBIN,), jnp.int32),
          pltpu.VMEM((NBIN,), jnp.int32),
          pltpu.VMEM((NBIN,), jnp.int32),
          pltpu.VMEM((NVR,), jnp.int32),
          pltpu.SMEM((NBIN + 2,), jnp.int32),
          pltpu.SMEM((NBIN + 2,), jnp.int32),
          pltpu.SMEM((8,), jnp.int32),
          pltpu.SemaphoreType.DMA,
          pltpu.SemaphoreType.DMA,
      ],
  )
  ug, vg = k1(user_table.T, item_table.T,
              user_table[TAIL0:], item_table[TAIL0:],
              user_indices, item_indices)
  k2 = pl.kernel(
      _dot_kernel,
      out_type=jax.ShapeDtypeStruct((B,), jnp.float32),
      mesh=mesh,
      compiler_params=pltpu.CompilerParams(needs_layout_passes=False),
      scratch_types=[
          pltpu.VMEM((DCH, GW), jnp.float32),
          pltpu.VMEM((DCH, GW), jnp.float32),
          pltpu.VMEM((L, L), jnp.float32),
          pltpu.VMEM((BPW,), jnp.float32),
          pltpu.SemaphoreType.DMA,
      ],
  )
  return k2(ug, vg)


def kernel(user_indices, item_indices, user_table, item_table):
  return _towers(user_indices.astype(jnp.int32),
                 item_indices.astype(jnp.int32),
                 user_table, item_table)


# speedup vs baseline: 9.3432x; 1.2222x over previous
import functools
import jax
import jax.numpy as jnp
from jax import lax
from jax.experimental import pallas as pl
from jax.experimental.pallas import tpu as pltpu
from jax.experimental.pallas import tpu_sc as plsc

NU = 1_000_000
B = 16384
D = 64
NW = 32
BPW = B // NW
L = 16
CPG = 128
SH = 7
NGW = 244
NGX = 4
NGMAX = 245
NBIN = 256
TAIL0 = (NW * NGW + NGX) * CPG
NVR = B // L
NSB = 4
GW = 128
DCH = 128


def _scalar(x):
  return x if jnp.ndim(x) == 0 else x[0]


def _splat(x):
  return x if jnp.ndim(x) else jnp.full((L,), x)


def _sweep(tbl, tailr, idx_hbm, out_hbm, idx_v, blk, stage, bidx,
           entry, scrv, cntv, runv, confv, cnt, off, misc, sem, semr,
           wid, lo, ng, lane_iota):
  hi_scan = jnp.where(wid == NW - 1, NU, lo + ng * CPG)
  m0 = lane_iota == 0
  ones = jnp.ones((L,), jnp.int32)

  pltpu.sync_copy(idx_hbm, idx_v)

  for k in range(NBIN // L):
    cntv[pl.ds(k * L, L)] = jnp.zeros((L,), jnp.int32)

  def fast_pass(apply_vec):
    def vbody(v, _):
      iv = idx_v[pl.ds(v * L, L)]
      m = (iv >= lo) & (iv < hi_scan)
      il = iv - lo
      g = jnp.where(m, il >> SH, NGMAX + 1)
      plsc.store_scatter(scrv, [g], lane_iota)
      rb = plsc.load_gather(scrv, [g])
      dup = m & (rb != lane_iota)
      dupc = _splat(plsc.all_reduce_population_count(dup))
      ok = m & (dupc == 0)
      apply_vec(g, il, v * L + lane_iota, ok)
      plsc.store_scatter(confv, [jnp.full((L,), v, jnp.int32)], dupc,
                         mask=m0)
      return 0

    lax.fori_loop(0, NVR, vbody, 0)

  def repair_pass(apply_vec):
    def qbody(q, _):
      cv = confv[pl.ds(q * L, L)]
      s = _scalar(plsc.all_reduce_population_count(cv != 0))

      @pl.when(s > 0)
      def _():
        def cond(mq):
          return _scalar(plsc.all_reduce_population_count(mq != 0)) > 0

        def wbody(mq):
          kq = _splat(plsc.all_reduce_ffs(mq != 0))
          v = q * L + _scalar(kq)
          iv = idx_v[pl.ds(v * L, L)]
          m2 = (iv >= lo) & (iv < hi_scan)
          il2 = iv - lo
          g2 = jnp.where(m2, il2 >> SH, NGMAX + 1)
          bv2 = v * L + lane_iota

          def cond2(mm):
            return _scalar(plsc.all_reduce_population_count(mm != 0)) > 0

          def wbody2(mm):
            k2 = _splat(plsc.all_reduce_ffs(mm != 0))
            sel = lane_iota == k2
            apply_vec(g2, il2, bv2, sel & m2)
            return jnp.where(sel, 0, mm)

          lax.while_loop(cond2, wbody2,
                         jnp.where(m2, jnp.int32(1), jnp.int32(0)))
          return jnp.where(lane_iota == kq, 0, mq)

        lax.while_loop(cond, wbody, cv)

      return 0

    lax.fori_loop(0, NVR // L, qbody, 0)

  def count_vec(g, il, bvec, ok):
    plsc.addupdate_scatter(cntv, [g], ones, mask=ok)

  fast_pass(count_vec)
  repair_pass(count_vec)

  tot = jnp.zeros((L,), jnp.int32)
  lastv = jnp.full((L,), L - 1, jnp.int32)
  for vb in range(NBIN // L):
    c = cntv[pl.ds(vb * L, L)]
    pc = plsc.cumsum(c)
    ex = tot + pc - c
    runv[pl.ds(vb * L, L)] = ex
    for k in range(L):
      off[vb * L + k] = ex[k]
      cnt[vb * L + k] = c[k]
    tot = tot + jnp.take(pc, lastv)

  def place_vec(g, il, bvec, ok):
    pos = plsc.load_gather(runv, [g])
    packed = (il << 14) | bvec
    plsc.store_scatter(entry, [pos], packed, mask=ok)
    plsc.addupdate_scatter(runv, [g], ones, mask=ok)

  fast_pass(place_vec)
  repair_pass(place_vec)

  misc[0] = 0

  def emit_begin():
    fc = misc[0]
    r = (fc >> 4) & (NSB - 1)
    p = fc & (L - 1)

    @pl.when((p == 0) & (fc >= NSB * L))
    def _():
      pltpu.make_async_copy(stage.at[0], out_hbm.at[bidx.at[0]],
                            semr).wait()

    return fc, r, p

  def emit_end(fc, r, p, bb):
    plsc.store_scatter(bidx.at[r], [jnp.full((L,), p, jnp.int32)],
                       jnp.full((L,), bb, jnp.int32), mask=m0)
    misc[0] = fc + 1

    @pl.when(p == L - 1)
    def _():
      pltpu.async_copy(stage.at[r], out_hbm.at[bidx.at[r]], semr)

  def issue_group(j, bf):
    pltpu.async_copy(tbl.at[:, pl.ds(lo + j * CPG, CPG)],
                     blk.at[bf], sem)

  @pl.when(cnt[0] > 0)
  def _():
    issue_group(0, 0)

  @pl.when((1 < ng) & (cnt[1] > 0))
  def _():
    issue_group(1, 1)

  def group_body(j, bj):
    @pl.when((j + 2 < ng) & (cnt[j + 2] > 0))
    def _():
      issue_group(j + 2, jnp.where(bj >= 1, bj - 1, bj + 2))

    @pl.when((j < ng) & (cnt[j] > 0))
    def _():
      pltpu.make_async_copy(tbl.at[:, pl.ds(0, CPG)], blk.at[0],
                            sem).wait()
      bufv = jnp.full((L,), bj, jnp.int32)
      e0 = off[j]
      e1 = off[j + 1]

      def entry_body(e, _):
        pa = (e >> 4) << 4
        pv = entry[pl.ds(pa, L)]
        pk = jnp.take(pv, jnp.full((L,), e - pa, jnp.int32))[0]
        il = pk >> 14
        bb = pk & 16383
        colv = jnp.full((L,), il - j * CPG, jnp.int32)
        fc, r, p = emit_begin()
        for cc in range(D // L):
          u = plsc.load_gather(blk, [bufv, lane_iota + cc * L, colv])
          stage[r, p, pl.ds(cc * L, L)] = u
        emit_end(fc, r, p, bb)
        return 0

      lax.fori_loop(e0, e1, entry_body, 0)
    return jnp.where(bj == 2, 0, bj + 1)

  lax.fori_loop(0, NGMAX, group_body, 0)

  @pl.when(wid == NW - 1)
  def _():
    pltpu.sync_copy(tailr, blk.at[0, pl.ds(0, 32)])
    e0 = off[NGW]
    e1 = off[NGW + 1]
    zv = jnp.zeros((L,), jnp.int32)

    def tail_entry(e, _):
      pa = (e >> 4) << 4
      pv = entry[pl.ds(pa, L)]
      pk = jnp.take(pv, jnp.full((L,), e - pa, jnp.int32))[0]
      il = pk >> 14
      bb = pk & 16383
      ul = il - NGW * CPG
      rowv = jnp.full((L,), ul >> 1, jnp.int32)
      cbase = (ul & 1) * D
      fc, r, p = emit_begin()
      for cc in range(D // L):
        colv = cbase + cc * L + lane_iota
        u = plsc.load_gather(blk, [zv, rowv, colv])
        stage[r, p, pl.ds(cc * L, L)] = u
      emit_end(fc, r, p, bb)
      return 0

    lax.fori_loop(e0, e1, tail_entry, 0)

  fcf = misc[0]
  ppf = fcf & (L - 1)
  rrf = (fcf >> 4) & (NSB - 1)

  @pl.when(ppf > 0)
  def _():
    brow = bidx[rrf, pl.ds(0, L)]
    brow = jnp.where(lane_iota >= ppf, B + lane_iota, brow)
    bidx[rrf, pl.ds(0, L)] = brow
    pltpu.async_copy(stage.at[rrf], out_hbm.at[bidx.at[rrf]], semr)

  ndrain = jnp.minimum((fcf + L - 1) >> 4, NSB)

  def drain_body(t, _):
    pltpu.make_async_copy(stage.at[0], out_hbm.at[bidx.at[0]], semr).wait()
    return 0
  lax.fori_loop(0, ndrain, drain_body, 0)


def _gather_kernel(ut, it, utail, itail, uidx, iidx, ug_hbm, vg_hbm,
                   idx_v, blk, stage, bidx, entry, scrv, cntv,
                   runv, confv, cnt, off, misc, sem, semr):
  wid = lax.axis_index("s") * 2 + lax.axis_index("c")
  lane_iota = lax.iota(jnp.int32, L)
  lo = (NGW * wid + jnp.minimum(wid, NGX)) * CPG
  ng = jnp.where(wid < NGX, NGW + 1, NGW)
  _sweep(ut, utail, uidx, ug_hbm, idx_v, blk, stage, bidx,
         entry, scrv, cntv, runv, confv, cnt, off, misc, sem, semr,
         wid, lo, ng, lane_iota)
  _sweep(it, itail, iidx, vg_hbm, idx_v, blk, stage, bidx,
         entry, scrv, cntv, runv, confv, cnt, off, misc, sem, semr,
         wid, lo, ng, lane_iota)


def _dot_kernel(ug_hbm, vg_hbm, out_hbm, uvm, vvm, pbuf, out_v, sem):
  wid = lax.axis_index("s") * 2 + lax.axis_index("c")
  base = wid * BPW
  row_iota = lax.iota(jnp.int32, L)

  def chunk_body(c, _):
    cb = pl.multiple_of(c * DCH, DCH)
    cu = pltpu.async_copy(ug_hbm.at[pl.ds(base + cb, DCH)], uvm, sem)
    cv = pltpu.async_copy(vg_hbm.at[pl.ds(base + cb, DCH)], vvm, sem)
    cu.wait()
    cv.wait()

    def group_body(g, _):
      row0 = pl.multiple_of(g * L, L)
      for k in range(L):
        r = row0 + k
        s = None
        for cc in range(D // L):
          u = uvm[r, pl.ds(cc * L, L)]
          v = vvm[r, pl.ds(cc * L, L)]
          m = u * v
          s = m if s is None else s + m
        pbuf[k, :] = s
      acc = jnp.zeros((L,), jnp.float32)
      for l in range(L):
        col = jnp.full((L,), l, jnp.int32)
        acc = acc + plsc.load_gather(pbuf, [row_iota, col])
      out_v[pl.ds(cb + row0, L)] = acc
      return 0

    lax.fori_loop(0, DCH // L, group_body, 0)
    return 0

  lax.fori_loop(0, BPW // DCH, chunk_body, 0)
  pltpu.sync_copy(out_v, out_hbm.at[pl.ds(base, BPW)])


@jax.jit
def _towers(user_indices, item_indices, user_table, item_table):
  mesh = plsc.VectorSubcoreMesh(core_axis_name="c", subcore_axis_name="s")
  k1 = pl.kernel(
      _gather_kernel,
      out_type=(pltpu.HBM((B + L, GW), jnp.float32),
                pltpu.HBM((B + L, GW), jnp.float32)),
      mesh=mesh,
      compiler_params=pltpu.CompilerParams(needs_layout_passes=False),
      scratch_types=[
          pltpu.VMEM((B,), jnp.int32),
          pltpu.VMEM((3, D, CPG), jnp.float32),
          pltpu.VMEM((NSB, L, GW), jnp.float32),
          pltpu.VMEM((NSB, L), jnp.int32),
          pltpu.VMEM((B + L,), jnp.int32),
          pltpu.VMEM((NBIN,), jnp.int32),
          pltpu.VMEM((NBIN,), jnp.int32),
          pltpu.VMEM((NBIN,), jnp.int32),
          pltpu.VMEM((NVR,), jnp.int32),
          pltpu.SMEM((NBIN + 2,), jnp.int32),
          pltpu.SMEM((NBIN + 2,), jnp.int32),
          pltpu.SMEM((8,), jnp.int32),
          pltpu.SemaphoreType.DMA,
          pltpu.SemaphoreType.DMA,
      ],
  )
  ug, vg = k1(user_table.T, item_table.T,
              user_table[TAIL0:].reshape(32, GW),
              item_table[TAIL0:].reshape(32, GW),
              user_indices, item_indices)
  k2 = pl.kernel(
      _dot_kernel,
      out_type=jax.ShapeDtypeStruct((B,), jnp.float32),
      mesh=mesh,
      compiler_params=pltpu.CompilerParams(needs_layout_passes=False),
      scratch_types=[
          pltpu.VMEM((DCH, GW), jnp.float32),
          pltpu.VMEM((DCH, GW), jnp.float32),
          pltpu.VMEM((L, L), jnp.float32),
          pltpu.VMEM((BPW,), jnp.float32),
          pltpu.SemaphoreType.DMA,
      ],
  )
  return k2(ug, vg)


def kernel(user_indices, item_indices, user_table, item_table):
  return _towers(user_indices.astype(jnp.int32),
                 item_indices.astype(jnp.int32),
                 user_table, item_table)


# speedup vs baseline: 9.3438x; 1.0001x over previous
import functools
import jax
import jax.numpy as jnp
from jax import lax
from jax.experimental import pallas as pl
from jax.experimental.pallas import tpu as pltpu
from jax.experimental.pallas import tpu_sc as plsc

NU = 1_000_000
B = 16384
D = 64
NW = 32
BPW = B // NW
L = 16
CPG = 128
SH = 7
NGW = 244
NGX = 4
NGMAX = 245
NBIN = 256
TAIL0 = (NW * NGW + NGX) * CPG
NVR = B // L
NSB = 4
GW = 128
DCH = 128


def _scalar(x):
  return x if jnp.ndim(x) == 0 else x[0]


def _splat(x):
  return x if jnp.ndim(x) else jnp.full((L,), x)


def _sweep(tbl, tailr, idx_hbm, out_hbm, idx_v, blk, stage, bidx,
           entry, scrv, cntv, runv, confv, cnt, off, misc, sem, semr,
           wid, lo, ng, lane_iota):
  hi_scan = jnp.where(wid == NW - 1, NU, lo + ng * CPG)
  m0 = lane_iota == 0
  ones = jnp.ones((L,), jnp.int32)

  pltpu.sync_copy(idx_hbm, idx_v)

  for k in range(NBIN // L):
    cntv[pl.ds(k * L, L)] = jnp.zeros((L,), jnp.int32)

  def fast_pass(apply_vec):
    def vbody(v4, _):
      for u in range(4):
        v = v4 * 4 + u
        iv = idx_v[pl.ds(v * L, L)]
        m = (iv >= lo) & (iv < hi_scan)
        il = iv - lo
        g = jnp.where(m, il >> SH, NGMAX + 1)
        plsc.store_scatter(scrv, [g], lane_iota)
        rb = plsc.load_gather(scrv, [g])
        dup = m & (rb != lane_iota)
        dupc = _splat(plsc.all_reduce_population_count(dup))
        ok = m & (dupc == 0)
        apply_vec(g, il, v * L + lane_iota, ok)
        plsc.store_scatter(confv, [jnp.full((L,), v, jnp.int32)], dupc,
                           mask=m0)
      return 0

    lax.fori_loop(0, NVR // 4, vbody, 0)

  def repair_pass(apply_vec):
    def qbody(q, _):
      cv = confv[pl.ds(q * L, L)]
      s = _scalar(plsc.all_reduce_population_count(cv != 0))

      @pl.when(s > 0)
      def _():
        def cond(mq):
          return _scalar(plsc.all_reduce_population_count(mq != 0)) > 0

        def wbody(mq):
          kq = _splat(plsc.all_reduce_ffs(mq != 0))
          v = q * L + _scalar(kq)
          iv = idx_v[pl.ds(v * L, L)]
          m2 = (iv >= lo) & (iv < hi_scan)
          il2 = iv - lo
          g2 = jnp.where(m2, il2 >> SH, NGMAX + 1)
          bv2 = v * L + lane_iota

          def cond2(mm):
            return _scalar(plsc.all_reduce_population_count(mm != 0)) > 0

          def wbody2(mm):
            k2 = _splat(plsc.all_reduce_ffs(mm != 0))
            sel = lane_iota == k2
            apply_vec(g2, il2, bv2, sel & m2)
            return jnp.where(sel, 0, mm)

          lax.while_loop(cond2, wbody2,
                         jnp.where(m2, jnp.int32(1), jnp.int32(0)))
          return jnp.where(lane_iota == kq, 0, mq)

        lax.while_loop(cond, wbody, cv)

      return 0

    lax.fori_loop(0, NVR // L, qbody, 0)

  def count_vec(g, il, bvec, ok):
    plsc.addupdate_scatter(cntv, [g], ones, mask=ok)

  fast_pass(count_vec)
  repair_pass(count_vec)

  tot = jnp.zeros((L,), jnp.int32)
  lastv = jnp.full((L,), L - 1, jnp.int32)
  for vb in range(NBIN // L):
    c = cntv[pl.ds(vb * L, L)]
    pc = plsc.cumsum(c)
    ex = tot + pc - c
    runv[pl.ds(vb * L, L)] = ex
    for k in range(L):
      off[vb * L + k] = ex[k]
      cnt[vb * L + k] = c[k]
    tot = tot + jnp.take(pc, lastv)

  def place_vec(g, il, bvec, ok):
    pos = plsc.load_gather(runv, [g])
    packed = (il << 14) | bvec
    plsc.store_scatter(entry, [pos], packed, mask=ok)
    plsc.addupdate_scatter(runv, [g], ones, mask=ok)

  fast_pass(place_vec)
  repair_pass(place_vec)

  misc[0] = 0

  def emit_begin():
    fc = misc[0]
    r = (fc >> 4) & (NSB - 1)
    p = fc & (L - 1)

    @pl.when((p == 0) & (fc >= NSB * L))
    def _():
      pltpu.make_async_copy(stage.at[0], out_hbm.at[bidx.at[0]],
                            semr).wait()

    return fc, r, p

  def emit_end(fc, r, p, bb):
    plsc.store_scatter(bidx.at[r], [jnp.full((L,), p, jnp.int32)],
                       jnp.full((L,), bb, jnp.int32), mask=m0)
    misc[0] = fc + 1

    @pl.when(p == L - 1)
    def _():
      pltpu.async_copy(stage.at[r], out_hbm.at[bidx.at[r]], semr)

  def issue_group(j, bf):
    pltpu.async_copy(tbl.at[:, pl.ds(lo + j * CPG, CPG)],
                     blk.at[bf], sem)

  @pl.when(cnt[0] > 0)
  def _():
    issue_group(0, 0)

  @pl.when((1 < ng) & (cnt[1] > 0))
  def _():
    issue_group(1, 1)

  def group_body(j, bj):
    @pl.when((j + 2 < ng) & (cnt[j + 2] > 0))
    def _():
      issue_group(j + 2, jnp.where(bj >= 1, bj - 1, bj + 2))

    @pl.when((j < ng) & (cnt[j] > 0))
    def _():
      pltpu.make_async_copy(tbl.at[:, pl.ds(0, CPG)], blk.at[0],
                            sem).wait()
      bufv = jnp.full((L,), bj, jnp.int32)
      e0 = off[j]
      e1 = off[j + 1]

      def entry_body(e, _):
        pa = (e >> 4) << 4
        pv = entry[pl.ds(pa, L)]
        pk = jnp.take(pv, jnp.full((L,), e - pa, jnp.int32))[0]
        il = pk >> 14
        bb = pk & 16383
        colv = jnp.full((L,), il - j * CPG, jnp.int32)
        fc, r, p = emit_begin()
        for cc in range(D // L):
          u = plsc.load_gather(blk, [bufv, lane_iota + cc * L, colv])
          stage[r, p, pl.ds(cc * L, L)] = u
        emit_end(fc, r, p, bb)
        return 0

      lax.fori_loop(e0, e1, entry_body, 0)
    return jnp.where(bj == 2, 0, bj + 1)

  lax.fori_loop(0, NGMAX, group_body, 0)

  @pl.when(wid == NW - 1)
  def _():
    pltpu.sync_copy(tailr, blk.at[0, pl.ds(0, 32)])
    e0 = off[NGW]
    e1 = off[NGW + 1]
    zv = jnp.zeros((L,), jnp.int32)

    def tail_entry(e, _):
      pa = (e >> 4) << 4
      pv = entry[pl.ds(pa, L)]
      pk = jnp.take(pv, jnp.full((L,), e - pa, jnp.int32))[0]
      il = pk >> 14
      bb = pk & 16383
      ul = il - NGW * CPG
      rowv = jnp.full((L,), ul >> 1, jnp.int32)
      cbase = (ul & 1) * D
      fc, r, p = emit_begin()
      for cc in range(D // L):
        colv = cbase + cc * L + lane_iota
        u = plsc.load_gather(blk, [zv, rowv, colv])
        stage[r, p, pl.ds(cc * L, L)] = u
      emit_end(fc, r, p, bb)
      return 0

    lax.fori_loop(e0, e1, tail_entry, 0)

  fcf = misc[0]
  ppf = fcf & (L - 1)
  rrf = (fcf >> 4) & (NSB - 1)

  @pl.when(ppf > 0)
  def _():
    brow = bidx[rrf, pl.ds(0, L)]
    brow = jnp.where(lane_iota >= ppf, B + lane_iota, brow)
    bidx[rrf, pl.ds(0, L)] = brow
    pltpu.async_copy(stage.at[rrf], out_hbm.at[bidx.at[rrf]], semr)

  ndrain = jnp.minimum((fcf + L - 1) >> 4, NSB)

  def drain_body(t, _):
    pltpu.make_async_copy(stage.at[0], out_hbm.at[bidx.at[0]], semr).wait()
    return 0
  lax.fori_loop(0, ndrain, drain_body, 0)


def _gather_kernel(ut, it, utail, itail, uidx, iidx, ug_hbm, vg_hbm,
                   idx_v, blk, stage, bidx, entry, scrv, cntv,
                   runv, confv, cnt, off, misc, sem, semr):
  wid = lax.axis_index("s") * 2 + lax.axis_index("c")
  lane_iota = lax.iota(jnp.int32, L)
  lo = (NGW * wid + jnp.minimum(wid, NGX)) * CPG
  ng = jnp.where(wid < NGX, NGW + 1, NGW)
  _sweep(ut, utail, uidx, ug_hbm, idx_v, blk, stage, bidx,
         entry, scrv, cntv, runv, confv, cnt, off, misc, sem, semr,
         wid, lo, ng, lane_iota)
  _sweep(it, itail, iidx, vg_hbm, idx_v, blk, stage, bidx,
         entry, scrv, cntv, runv, confv, cnt, off, misc, sem, semr,
         wid, lo, ng, lane_iota)


def _dot_kernel(ug_hbm, vg_hbm, out_hbm, uvm, vvm, pbuf, out_v, sem):
  wid = lax.axis_index("s") * 2 + lax.axis_index("c")
  base = wid * BPW
  row_iota = lax.iota(jnp.int32, L)

  def chunk_body(c, _):
    cb = pl.multiple_of(c * DCH, DCH)
    cu = pltpu.async_copy(ug_hbm.at[pl.ds(base + cb, DCH)], uvm, sem)
    cv = pltpu.async_copy(vg_hbm.at[pl.ds(base + cb, DCH)], vvm, sem)
    cu.wait()
    cv.wait()

    def group_body(g, _):
      row0 = pl.multiple_of(g * L, L)
      for k in range(L):
        r = row0 + k
        s = None
        for cc in range(D // L):
          u = uvm[r, pl.ds(cc * L, L)]
          v = vvm[r, pl.ds(cc * L, L)]
          m = u * v
          s = m if s is None else s + m
        pbuf[k, :] = s
      acc = jnp.zeros((L,), jnp.float32)
      for l in range(L):
        col = jnp.full((L,), l, jnp.int32)
        acc = acc + plsc.load_gather(pbuf, [row_iota, col])
      out_v[pl.ds(cb + row0, L)] = acc
      return 0

    lax.fori_loop(0, DCH // L, group_body, 0)
    return 0

  lax.fori_loop(0, BPW // DCH, chunk_body, 0)
  pltpu.sync_copy(out_v, out_hbm.at[pl.ds(base, BPW)])


@jax.jit
def _towers(user_indices, item_indices, user_table, item_table):
  mesh = plsc.VectorSubcoreMesh(core_axis_name="c", subcore_axis_name="s")
  k1 = pl.kernel(
      _gather_kernel,
      out_type=(pltpu.HBM((B + L, GW), jnp.float32),
                pltpu.HBM((B + L, GW), jnp.float32)),
      mesh=mesh,
      compiler_params=pltpu.CompilerParams(needs_layout_passes=False),
      scratch_types=[
          pltpu.VMEM((B,), jnp.int32),
          pltpu.VMEM((3, D, CPG), jnp.float32),
          pltpu.VMEM((NSB, L, GW), jnp.float32),
          pltpu.VMEM((NSB, L), jnp.int32),
          pltpu.VMEM((B + L,), jnp.int32),
          pltpu.VMEM((NBIN,), jnp.int32),
          pltpu.VMEM((NBIN,), jnp.int32),
          pltpu.VMEM((NBIN,), jnp.int32),
          pltpu.VMEM((NVR,), jnp.int32),
          pltpu.SMEM((NBIN + 2,), jnp.int32),
          pltpu.SMEM((NBIN + 2,), jnp.int32),
          pltpu.SMEM((8,), jnp.int32),
          pltpu.SemaphoreType.DMA,
          pltpu.SemaphoreType.DMA,
      ],
  )
  ug, vg = k1(user_table.T, item_table.T,
              user_table[TAIL0:].reshape(32, GW),
              item_table[TAIL0:].reshape(32, GW),
              user_indices, item_indices)
  k2 = pl.kernel(
      _dot_kernel,
      out_type=jax.ShapeDtypeStruct((B,), jnp.float32),
      mesh=mesh,
      compiler_params=pltpu.CompilerParams(needs_layout_passes=False),
      scratch_types=[
          pltpu.VMEM((DCH, GW), jnp.float32),
          pltpu.VMEM((DCH, GW), jnp.float32),
          pltpu.VMEM((L, L), jnp.float32),
          pltpu.VMEM((BPW,), jnp.float32),
          pltpu.SemaphoreType.DMA,
      ],
  )
  return k2(ug, vg)


def kernel(user_indices, item_indices, user_table, item_table):
  return _towers(user_indices.astype(jnp.int32),
                 item_indices.astype(jnp.int32),
                 user_table, item_table)
